# baseline (speedup 1.0000x reference)
import functools

import jax
import jax.numpy as jnp
from jax import lax
from jax.experimental import pallas as pl
from jax.experimental.pallas import tpu as pltpu
from jax.experimental.pallas import tpu_sc as plsc

NUM_BINS = 256
N_IMG = 16
W = 512
ROWS = 64
PIX = 3 * 512 * W
HALF = PIX // 2
CH = ROWS * W
NCH = HALF // CH
VPR = W // 16
LANES = 16
NWORKERS = 32
HIST = NUM_BINS * LANES


def _sc_histograms(im1, im2):
    mesh = plsc.VectorSubcoreMesh(core_axis_name="c", subcore_axis_name="s",
                                  num_cores=2, num_subcores=16)

    @functools.partial(
        pl.kernel,
        mesh=mesh,
        compiler_params=pltpu.CompilerParams(needs_layout_passes=False,
                                             use_tc_tiling_on_sc=True),
        out_type=jax.ShapeDtypeStruct((NWORKERS,), jnp.float32),
        scratch_types=[
            pltpu.VMEM((ROWS, W), jnp.float32),
            pltpu.VMEM((ROWS, W), jnp.float32),
            pltpu.VMEM((ROWS, W), jnp.float32),
            pltpu.VMEM((HIST,), jnp.float32),
            pltpu.VMEM((HIST,), jnp.float32),
            pltpu.VMEM((NUM_BINS,), jnp.float32),
            pltpu.VMEM((2 * LANES * NUM_BINS,), jnp.float32),
            pltpu.VMEM_SHARED((2 * LANES * NUM_BINS,), jnp.float32),
            pltpu.SemaphoreType.DMA,
            pltpu.SemaphoreType.DMA,
            pltpu.SemaphoreType.DMA,
        ],
    )
    def hist_kernel(src1, src2, out, buf0, buf1, buf2, hist1, hist2, stage,
                    fin, shared, sem0, sem1, sem2):
        wid = lax.axis_index("c") * 16 + lax.axis_index("s")
        img = wid // 2
        parity = wid % 2
        lane = lax.iota(jnp.int32, LANES)
        lane_u = lax.iota(jnp.uint32, LANES)
        ones = jnp.ones((LANES,), jnp.float32)
        zeros = jnp.zeros((LANES,), jnp.float32)
        bufs = (buf0, buf1, buf2)
        sems = (sem0, sem1, sem2)
        NBUF = len(bufs)
        gather_base = lane * LANES

        def zero_body(j, _):
            hist1[pl.ds(j * LANES, LANES)] = zeros
            hist2[pl.ds(j * LANES, LANES)] = zeros
            return 0

        def run_chunk(buf, hist):
            def chunk_body(i):
                x = buf[i // VPR, pl.ds((i % VPR) * LANES, LANES)]
                b = plsc.bitcast(x + 1.0, jnp.uint32)
                addr = ((b >> jnp.uint32(11)) & jnp.uint32(0xFF0)) | lane_u
                plsc.addupdate_scatter(hist, [plsc.bitcast(addr, jnp.int32)],
                                       ones)
            plsc.parallel_loop(0, CH // LANES, unroll=8)(chunk_body)

        def merge(hist, shared_row):
            for j in range(LANES):
                acc = zeros
                for l in range(LANES):
                    acc = acc + plsc.load_gather(
                        hist, [gather_base + (j * NUM_BINS + l)])
                stage[pl.ds(j * LANES, LANES)] = acc
            pltpu.sync_copy(stage,
                            shared.at[pl.ds(shared_row * NUM_BINS, NUM_BINS)])

        lax.fori_loop(0, NUM_BINS, zero_body, 0)

        jobs = [(src1, hist1, g) for g in range(NCH)] + \
               [(src2, hist2, g) for g in range(NCH)]

        def start(k):
            src, _, g = jobs[k]
            glob = parity * NCH + g
            chan, r0 = glob // 8, (glob % 8) * ROWS
            return pltpu.async_copy(
                src.at[img, chan, pl.ds(r0, ROWS), :],
                bufs[k % NBUF], sems[k % NBUF])

        handles = {k: start(k) for k in range(NBUF)}
        for k in range(len(jobs)):
            handles.pop(k).wait()
            run_chunk(bufs[k % NBUF], jobs[k][1])
            if k + NBUF < len(jobs):
                handles[k + NBUF] = start(k + NBUF)

        sid = lax.axis_index("s")
        merge(hist1, sid)
        merge(hist2, sid + LANES)
        plsc.subcore_barrier()

        @pl.when(sid == 0)
        def _finalize():
            pltpu.sync_copy(shared, fin)
            inv_n = jnp.float32(1.0 / PIX)
            emd = zeros
            for j in range(8):
                carry = jnp.float32(0.0)
                r1a, r1b = 2 * j * NUM_BINS, (2 * j + 1) * NUM_BINS
                r2a = (2 * LANES // 2 + 2 * j) * NUM_BINS
                r2b = r2a + NUM_BINS
                for q in range(NUM_BINS // LANES):
                    o = q * LANES
                    d = (fin[pl.ds(r1a + o, LANES)]
                         + fin[pl.ds(r1b + o, LANES)]
                         - fin[pl.ds(r2a + o, LANES)]
                         - fin[pl.ds(r2b + o, LANES)]) * inv_n
                    c = plsc.cumsum(d) + carry
                    emd = emd + jnp.abs(c)
                    carry = carry + jnp.sum(d)
            emd = emd / jnp.float32(NUM_BINS) / jnp.float32(3.0)
            total = jnp.sum(emd)
            stage[pl.ds(0, LANES)] = jnp.where(lane == 0, total, 0.0)
            cidx = lax.axis_index("c")
            pltpu.sync_copy(stage.at[pl.ds(0, LANES)],
                            out.at[pl.ds(cidx * LANES, LANES)])

    return hist_kernel(im1, im2)


def kernel(im1, im2):
    parts = _sc_histograms(im1, im2)
    return parts[0] + parts[LANES]


# speedup vs baseline: 1.0428x; 1.0428x over previous
import functools

import jax
import jax.numpy as jnp
from jax import lax
from jax.experimental import pallas as pl
from jax.experimental.pallas import tpu as pltpu
from jax.experimental.pallas import tpu_sc as plsc

NUM_BINS = 256
N_IMG = 16
W = 512
ROWS = 64
PIX = 3 * 512 * W
HALF = PIX // 2
CH = ROWS * W
NCH = HALF // CH
VPR = W // 16
LANES = 16
NWORKERS = 32
HIST = NUM_BINS * LANES


def _sc_histograms(im1, im2):
    mesh = plsc.VectorSubcoreMesh(core_axis_name="c", subcore_axis_name="s",
                                  num_cores=2, num_subcores=16)

    @functools.partial(
        pl.kernel,
        mesh=mesh,
        compiler_params=pltpu.CompilerParams(needs_layout_passes=False,
                                             use_tc_tiling_on_sc=True),
        out_type=jax.ShapeDtypeStruct((2 * NWORKERS * NUM_BINS,), jnp.float32),
        scratch_types=[
            pltpu.VMEM((ROWS, W), jnp.float32),
            pltpu.VMEM((ROWS, W), jnp.float32),
            pltpu.VMEM((ROWS, W), jnp.float32),
            pltpu.VMEM((HIST,), jnp.float32),
            pltpu.VMEM((HIST,), jnp.float32),
            pltpu.VMEM((NUM_BINS,), jnp.float32),
            pltpu.SemaphoreType.DMA,
            pltpu.SemaphoreType.DMA,
            pltpu.SemaphoreType.DMA,
        ],
    )
    def hist_kernel(src1, src2, out, buf0, buf1, buf2, hist1, hist2, stage,
                    sem0, sem1, sem2):
        wid = lax.axis_index("c") * 16 + lax.axis_index("s")
        img = wid // 2
        parity = wid % 2
        lane = lax.iota(jnp.int32, LANES)
        lane_u = lax.iota(jnp.uint32, LANES)
        ones = jnp.ones((LANES,), jnp.float32)
        zeros = jnp.zeros((LANES,), jnp.float32)
        bufs = (buf0, buf1, buf2)
        sems = (sem0, sem1, sem2)
        NBUF = len(bufs)
        gather_base = lane * LANES

        def zero_body(j, _):
            hist1[pl.ds(j * LANES, LANES)] = zeros
            hist2[pl.ds(j * LANES, LANES)] = zeros
            return 0

        def run_chunk(buf, hist):
            def chunk_body(i):
                x = buf[i // VPR, pl.ds((i % VPR) * LANES, LANES)]
                b = plsc.bitcast(x + 1.0, jnp.uint32)
                addr = ((b >> jnp.uint32(11)) & jnp.uint32(0xFF0)) | lane_u
                plsc.addupdate_scatter(hist, [plsc.bitcast(addr, jnp.int32)],
                                       ones)
            plsc.parallel_loop(0, CH // LANES, unroll=8)(chunk_body)

        def merge(hist, out_row):
            for j in range(LANES):
                acc = zeros
                for l in range(LANES):
                    acc = acc + plsc.load_gather(
                        hist, [gather_base + (j * NUM_BINS + l)])
                stage[pl.ds(j * LANES, LANES)] = acc
            pltpu.sync_copy(stage, out.at[pl.ds(out_row * NUM_BINS, NUM_BINS)])

        jobs = [(src1, hist1, g) for g in range(NCH)] + \
               [(src2, hist2, g) for g in range(NCH)]

        def start(k):
            src, _, g = jobs[k]
            glob = parity * NCH + g
            chan, r0 = glob // 8, (glob % 8) * ROWS
            return pltpu.async_copy(
                src.at[img, chan, pl.ds(r0, ROWS), :],
                bufs[k % NBUF], sems[k % NBUF])

        handles = {k: start(k) for k in range(NBUF)}
        lax.fori_loop(0, NUM_BINS, zero_body, 0)
        for k in range(len(jobs)):
            handles.pop(k).wait()
            run_chunk(bufs[k % NBUF], jobs[k][1])
            if k + NBUF < len(jobs):
                handles[k + NBUF] = start(k + NBUF)

        row = parity * N_IMG + img
        merge(hist1, row)
        merge(hist2, row + NWORKERS)

    return hist_kernel(im1, im2)


def _tc_reduce(parts):
    def body(p_ref, out_ref):
        p = p_ref[...]
        h1 = (p[0:16] + p[16:32]) / float(PIX)
        h2 = (p[32:48] + p[48:64]) / float(PIX)
        r = lax.broadcasted_iota(jnp.int32, (NUM_BINS, NUM_BINS), 0)
        c = lax.broadcasted_iota(jnp.int32, (NUM_BINS, NUM_BINS), 1)
        tri = (r <= c).astype(jnp.float32)
        c1 = jnp.dot(h1, tri, preferred_element_type=jnp.float32,
                     precision=lax.Precision.HIGHEST)
        c2 = jnp.dot(h2, tri, preferred_element_type=jnp.float32,
                     precision=lax.Precision.HIGHEST)
        emd = jnp.sum(jnp.abs(c1 - c2))
        out_ref[0, 0] = emd / float(NUM_BINS) / 3.0

    return pl.pallas_call(
        body,
        out_shape=jax.ShapeDtypeStruct((1, 1), jnp.float32),
        out_specs=pl.BlockSpec(memory_space=pltpu.SMEM),
    )(parts)


def kernel(im1, im2):
    parts = _sc_histograms(im1, im2)
    out = _tc_reduce(parts.reshape(2 * NWORKERS, NUM_BINS))
    return out[0, 0]
